# baseline (speedup 1.0000x reference)
import math

import jax
import jax.numpy as jnp
from jax import lax
from jax.experimental import pallas as pl
from jax.experimental.pallas import tpu as pltpu
from jax.experimental.pallas import tpu_sc as plsc

_LANES = 16
_CH = 16
_NB = 6


def _scale_tc(x, scale):
    B, S, D = x.shape
    xs = x.reshape(B * S, D)
    rows = B * S
    blk = 512

    def body(x_ref, o_ref):
        o_ref[...] = x_ref[...] * scale

    out = pl.pallas_call(
        body,
        out_shape=jax.ShapeDtypeStruct((rows, D), jnp.float32),
        grid=(rows // blk,),
        in_specs=[pl.BlockSpec((blk, D), lambda i: (i, 0))],
        out_specs=pl.BlockSpec((blk, D), lambda i: (i, 0)),
    )(xs)
    return out.reshape(B, S, D)


def _pe_gather_sc(pe2d, offset, B, S, D):
    info = plsc.get_sparse_core_info()
    NW = info.num_cores * info.num_subcores
    NC = info.num_cores
    cpb = NW // B
    rpw = S // cpb
    nch = rpw // _CH

    mesh = plsc.VectorSubcoreMesh(core_axis_name="c", subcore_axis_name="s")

    def body(pe_hbm, off_hbm, out_hbm, off_v, idx_v, buf, *sems):
        ld_sems, st_sems = sems[:_NB], sems[_NB:]
        wid = lax.axis_index("s") * NC + lax.axis_index("c")
        b = wid // cpb
        base = (wid % cpb) * rpw
        outbase = wid * rpw

        pltpu.sync_copy(off_hbm, off_v)
        off_b = plsc.load_gather(off_v, [jnp.full((_LANES,), b, jnp.int32)])
        lanes = lax.iota(jnp.int32, _LANES)

        def fill_idx(p, j):
            for k in range(_CH // _LANES):
                idx_v[p, pl.ds(k * _LANES, _LANES)] = (
                    off_b + (base + j * _CH + k * _LANES) + lanes
                )

        ld = [None] * nch
        st = [None] * nch
        for j in range(min(_NB, nch)):
            p = j % _NB
            fill_idx(p, j)
            ld[j] = pltpu.async_copy(pe_hbm.at[idx_v.at[p]], buf.at[p], ld_sems[p])
        for j in range(nch):
            p = j % _NB
            ld[j].wait()
            st[j] = pltpu.async_copy(
                buf.at[p], out_hbm.at[pl.ds(outbase + j * _CH, _CH), :], st_sems[p]
            )
            nj = j + _NB
            if nj < nch:
                st[j].wait()
                fill_idx(p, nj)
                ld[nj] = pltpu.async_copy(
                    pe_hbm.at[idx_v.at[p]], buf.at[p], ld_sems[p]
                )
        for j in range(max(0, nch - _NB), nch):
            st[j].wait()

    return pl.kernel(
        body,
        out_type=jax.ShapeDtypeStruct((B * S, D), jnp.float32),
        mesh=mesh,
        scratch_types=[
            pltpu.VMEM((B,), jnp.int32),
            pltpu.VMEM((_NB, _CH), jnp.int32),
            pltpu.VMEM((_NB, _CH, D), jnp.float32),
        ]
        + [pltpu.SemaphoreType.DMA] * (2 * _NB),
        compiler_params=pltpu.CompilerParams(needs_layout_passes=False),
    )(pe2d, offset)


def kernel(x, offset, pe):
    B, S, D = x.shape
    scale = math.sqrt(D)
    pe2d = pe[0]
    pos_emb = _pe_gather_sc(pe2d, offset, B, S, D).reshape(B, S, D)
    x_scaled = _scale_tc(x, scale)
    return (x_scaled, pos_emb)


# speedup vs baseline: 1.0016x; 1.0016x over previous
import math

import jax
import jax.numpy as jnp
from jax import lax
from jax.experimental import pallas as pl
from jax.experimental.pallas import tpu as pltpu
from jax.experimental.pallas import tpu_sc as plsc

_LANES = 16
_CH = 16
_NB = 6


def _scale_tc(x, scale):
    B, S, D = x.shape
    xs = x.reshape(B * S, D)
    rows = B * S
    blk = 512

    def body(x_ref, o_ref):
        o_ref[...] = x_ref[...] * scale

    out = pl.pallas_call(
        body,
        out_shape=jax.ShapeDtypeStruct((rows, D), jnp.float32),
        grid=(rows // blk,),
        in_specs=[pl.BlockSpec((blk, D), lambda i: (i, 0))],
        out_specs=pl.BlockSpec((blk, D), lambda i: (i, 0)),
    )(xs)
    return out.reshape(B, S, D)


def _pe_gather_sc(pe2d, offset, B, S, D):
    info = plsc.get_sparse_core_info()
    NW = info.num_cores * info.num_subcores
    NC = info.num_cores
    cpb = NW // B
    rpw = S // cpb
    nch = rpw // _CH

    mesh = plsc.VectorSubcoreMesh(core_axis_name="c", subcore_axis_name="s")

    def body(pe_hbm, off_hbm, out_hbm, off_v, idx_v, buf, *sems):
        ld_sems, st_sems = sems[:_NB], sems[_NB:]
        wid = lax.axis_index("s") * NC + lax.axis_index("c")
        b = wid // cpb
        base = (wid % cpb) * rpw
        outbase = wid * rpw

        pltpu.sync_copy(off_hbm, off_v)
        off_b = plsc.load_gather(off_v, [jnp.full((_LANES,), b, jnp.int32)])
        lanes = lax.iota(jnp.int32, _LANES)

        def fill_idx(p, j):
            for k in range(_CH // _LANES):
                idx_v[p, pl.ds(k * _LANES, _LANES)] = (
                    off_b + (base + j * _CH + k * _LANES) + lanes
                )

        K = _NB // 2
        ld = [None] * nch
        st = [None] * nch

        def issue_ld(j):
            p = j % _NB
            fill_idx(p, j)
            ld[j] = pltpu.async_copy(pe_hbm.at[idx_v.at[p]], buf.at[p], ld_sems[p])

        for j in range(min(K, nch)):
            issue_ld(j)
        for j in range(nch):
            p = j % _NB
            nj = j + K
            if nj < nch:
                onj = nj - _NB
                if onj >= 0:
                    st[onj].wait()
                issue_ld(nj)
            ld[j].wait()
            st[j] = pltpu.async_copy(
                buf.at[p], out_hbm.at[pl.ds(outbase + j * _CH, _CH), :], st_sems[p]
            )
        for j in range(max(0, nch - _NB), nch):
            st[j].wait()

    return pl.kernel(
        body,
        out_type=jax.ShapeDtypeStruct((B * S, D), jnp.float32),
        mesh=mesh,
        scratch_types=[
            pltpu.VMEM((B,), jnp.int32),
            pltpu.VMEM((_NB, _CH), jnp.int32),
            pltpu.VMEM((_NB, _CH, D), jnp.float32),
        ]
        + [pltpu.SemaphoreType.DMA] * (2 * _NB),
        compiler_params=pltpu.CompilerParams(needs_layout_passes=False),
    )(pe2d, offset)


def kernel(x, offset, pe):
    B, S, D = x.shape
    scale = math.sqrt(D)
    pe2d = pe[0]
    pos_emb = _pe_gather_sc(pe2d, offset, B, S, D).reshape(B, S, D)
    x_scaled = _scale_tc(x, scale)
    return (x_scaled, pos_emb)


# speedup vs baseline: 1.0217x; 1.0201x over previous
import math

import jax
import jax.numpy as jnp
from jax import lax
from jax.experimental import pallas as pl
from jax.experimental.pallas import tpu as pltpu
from jax.experimental.pallas import tpu_sc as plsc

_LANES = 16
_CH = 16
_NB = 6


def _scale_tc(x, scale):
    B, S, D = x.shape
    xs = x.reshape(B * S, D)
    rows = B * S
    blk = 1024

    def body(x_ref, o_ref):
        o_ref[...] = x_ref[...] * scale

    out = pl.pallas_call(
        body,
        out_shape=jax.ShapeDtypeStruct((rows, D), jnp.float32),
        grid=(rows // blk,),
        in_specs=[pl.BlockSpec((blk, D), lambda i: (i, 0))],
        out_specs=pl.BlockSpec((blk, D), lambda i: (i, 0)),
    )(xs)
    return out.reshape(B, S, D)


def _pe_gather_sc(pe2d, offset, B, S, D):
    info = plsc.get_sparse_core_info()
    NW = info.num_cores * info.num_subcores
    NC = info.num_cores
    cpb = NW // B
    rpw = S // cpb
    nch = rpw // _CH

    mesh = plsc.VectorSubcoreMesh(core_axis_name="c", subcore_axis_name="s")

    def body(pe_hbm, off_hbm, out_hbm, off_v, idx_v, buf, *sems):
        ld_sems, st_sems = sems[:_NB], sems[_NB:]
        wid = lax.axis_index("s") * NC + lax.axis_index("c")
        b = wid // cpb
        base = (wid % cpb) * rpw
        outbase = wid * rpw

        pltpu.sync_copy(off_hbm, off_v)
        off_b = plsc.load_gather(off_v, [jnp.full((_LANES,), b, jnp.int32)])
        lanes = lax.iota(jnp.int32, _LANES)

        def fill_idx(p, j):
            for k in range(_CH // _LANES):
                idx_v[p, pl.ds(k * _LANES, _LANES)] = (
                    off_b + (base + j * _CH + k * _LANES) + lanes
                )

        K = _NB // 2
        ld = [None] * nch
        st = [None] * nch

        def issue_ld(j):
            p = j % _NB
            fill_idx(p, j)
            ld[j] = pltpu.async_copy(pe_hbm.at[idx_v.at[p]], buf.at[p], ld_sems[p])

        for j in range(min(K, nch)):
            issue_ld(j)
        for j in range(nch):
            p = j % _NB
            nj = j + K
            if nj < nch:
                onj = nj - _NB
                if onj >= 0:
                    st[onj].wait()
                issue_ld(nj)
            ld[j].wait()
            st[j] = pltpu.async_copy(
                buf.at[p], out_hbm.at[pl.ds(outbase + j * _CH, _CH), :], st_sems[p]
            )
        for j in range(max(0, nch - _NB), nch):
            st[j].wait()

    return pl.kernel(
        body,
        out_type=jax.ShapeDtypeStruct((B * S, D), jnp.float32),
        mesh=mesh,
        scratch_types=[
            pltpu.VMEM((B,), jnp.int32),
            pltpu.VMEM((_NB, _CH), jnp.int32),
            pltpu.VMEM((_NB, _CH, D), jnp.float32),
        ]
        + [pltpu.SemaphoreType.DMA] * (2 * _NB),
        compiler_params=pltpu.CompilerParams(needs_layout_passes=False),
    )(pe2d, offset)


def kernel(x, offset, pe):
    B, S, D = x.shape
    scale = math.sqrt(D)
    pe2d = pe[0]
    pos_emb = _pe_gather_sc(pe2d, offset, B, S, D).reshape(B, S, D)
    x_scaled = _scale_tc(x, scale)
    return (x_scaled, pos_emb)
